# speedup vs baseline: 3.0447x; 3.0447x over previous
import functools

import jax
import jax.numpy as jnp
from jax import lax
from jax.experimental import pallas as pl
from jax.experimental.pallas import tpu as pltpu
from jax.experimental.pallas import tpu_sc as plsc

_D = 128
_FEW = 64
_NEIGH = 200
_NPAD = 208
_NSYM = 100000
_QROWS = 8192
_NW = 32
_QPW = _QROWS // _NW
_SEGS = 2 * _FEW
_SPW = _SEGS // _NW


def _sc_gather(qidx, sidx, table):
    mesh = plsc.VectorSubcoreMesh(core_axis_name="c", subcore_axis_name="s")

    @functools.partial(
        pl.kernel,
        out_type=(
            jax.ShapeDtypeStruct((_QROWS, _D), jnp.float32),
            jax.ShapeDtypeStruct((_SEGS, _D), jnp.float32),
        ),
        mesh=mesh,
        scratch_types=[
            pltpu.VMEM((_QPW,), jnp.int32),
            pltpu.VMEM((_QPW, _D), jnp.float32),
            pltpu.VMEM((_SPW, _NPAD), jnp.int32),
            pltpu.VMEM((_NPAD, _D), jnp.float32),
            pltpu.VMEM((_SPW, _D), jnp.float32),
            pltpu.SemaphoreType.DMA,
        ],
    )
    def k(qidx_hbm, sidx_hbm, table_hbm, qrows_hbm, ssum_hbm,
          qidx_v, qrows_v, sidx_v, srows_v, acc_v, sem):
        wid = lax.axis_index("s") * 2 + lax.axis_index("c")
        qbase = wid * _QPW
        pltpu.sync_copy(qidx_hbm.at[pl.ds(qbase, _QPW)], qidx_v)
        for ch in range(_QPW // 128):
            pltpu.async_copy(
                table_hbm.at[qidx_v.at[pl.ds(ch * 128, 128)]],
                qrows_v.at[pl.ds(ch * 128, 128)], sem).wait()
        pltpu.sync_copy(qrows_v, qrows_hbm.at[pl.ds(qbase, _QPW)])
        sbase = wid * _SPW
        pltpu.sync_copy(sidx_hbm.at[pl.ds(sbase, _SPW)], sidx_v)
        for t in range(_SPW):
            pltpu.async_copy(
                table_hbm.at[sidx_v.at[t, pl.ds(0, 128)]],
                srows_v.at[pl.ds(0, 128)], sem).wait()
            pltpu.async_copy(
                table_hbm.at[sidx_v.at[t, pl.ds(128, _NPAD - 128)]],
                srows_v.at[pl.ds(128, _NPAD - 128)], sem).wait()

            def body(r, accs):
                return tuple(a + srows_v[r, pl.ds(kk * 16, 16)]
                             for kk, a in enumerate(accs))

            accs = lax.fori_loop(
                0, _NPAD, body,
                tuple(jnp.zeros((16,), jnp.float32) for _ in range(_D // 16)))
            for kk in range(_D // 16):
                acc_v[t, pl.ds(kk * 16, 16)] = accs[kk]
        pltpu.sync_copy(acc_v, ssum_hbm.at[pl.ds(sbase, _SPW)])

    return k(qidx, sidx, table)


def _support_body(ssum_ref, gcn_ref, p1_ref, p2_ref, gcnb_ref, p1b_ref,
                  p2b_ref, lna_ref, lnb_ref, sg_ref, msg_ref):
    ssum = ssum_ref[:]
    cat = jnp.concatenate([ssum[:_FEW], ssum[_FEW:]], axis=1)
    out = lax.dot_general(cat, gcn_ref[:], (((1,), (1,)), ((), ())),
                          preferred_element_type=jnp.float32)
    out = out + gcnb_ref[:] * float(_NEIGH)
    support = jnp.tanh(out * (1.0 / _FEW))
    h = lax.dot_general(support, p1_ref[:], (((1,), (1,)), ((), ())),
                        preferred_element_type=jnp.float32) + p1b_ref[:]
    h = jnp.maximum(h, 0.0)
    h2 = lax.dot_general(h, p2_ref[:], (((1,), (1,)), ((), ())),
                         preferred_element_type=jnp.float32) + p2b_ref[:]
    z = h2 + support
    mu = jnp.mean(z, axis=1, keepdims=True)
    zc = z - mu
    sigma = jnp.sqrt(jnp.sum(zc * zc, axis=1, keepdims=True) / (_D - 1.0))
    sg = lna_ref[:] * zc / (sigma + 1e-3) + lnb_ref[:]
    sg_ref[:] = sg
    msg_ref[:] = jnp.mean(sg, axis=0, keepdims=True)


def _tc_support(ssum, gcn_w_w, proj1_w, proj2_w, gcn_b2, p1b2, p2b2,
                lna2, lnb2):
    return pl.pallas_call(
        _support_body,
        out_shape=(
            jax.ShapeDtypeStruct((_FEW, _D), jnp.float32),
            jax.ShapeDtypeStruct((1, _D), jnp.float32),
        ),
    )(ssum, gcn_w_w, proj1_w, proj2_w, gcn_b2, p1b2, p2b2, lna2, lnb2)


_BLK = 1024


def _lstm_body(x_ref, wih_ref, whh_ref, sg_ref, msg_ref, bg_ref, out_ref):
    x = x_ref[:]
    sg = sg_ref[:]
    gates_x = lax.dot_general(x, wih_ref[:], (((1,), (1,)), ((), ())),
                              preferred_element_type=jnp.float32) + bg_ref[:]

    def attn(hq):
        logits = lax.dot_general(hq, sg, (((1,), (1,)), ((), ())),
                                 preferred_element_type=jnp.float32)
        m = jnp.max(logits, axis=1, keepdims=True)
        e = jnp.exp(logits - m)
        a = e / jnp.sum(e, axis=1, keepdims=True)
        r = lax.dot_general(a, sg, (((1,), (0,)), ((), ())),
                            preferred_element_type=jnp.float32)
        return r

    g = gates_x
    c = jax.nn.sigmoid(g[:, 0:256]) * jnp.tanh(g[:, 512:768])
    h = jax.nn.sigmoid(g[:, 768:1024]) * jnp.tanh(c)
    hq = x + h[:, 0:_D]
    h_r = jnp.concatenate([hq, attn(hq)], axis=1)
    for step in range(1, 4):
        g = gates_x + lax.dot_general(h_r, whh_ref[:],
                                      (((1,), (1,)), ((), ())),
                                      preferred_element_type=jnp.float32)
        c = (jax.nn.sigmoid(g[:, 256:512]) * c
             + jax.nn.sigmoid(g[:, 0:256]) * jnp.tanh(g[:, 512:768]))
        h = jax.nn.sigmoid(g[:, 768:1024]) * jnp.tanh(c)
        hq = x + h[:, 0:_D]
        if step < 3:
            h_r = jnp.concatenate([hq, attn(hq)], axis=1)
    out_ref[:] = lax.dot_general(hq, msg_ref[:], (((1,), (1,)), ((), ())),
                                 preferred_element_type=jnp.float32)


def _tc_lstm(qrows, W_ih, W_hh, sg, msg, bg2):
    nblk = _QROWS // _BLK
    return pl.pallas_call(
        _lstm_body,
        grid=(nblk,),
        in_specs=[
            pl.BlockSpec((_BLK, _D), lambda i: (i, 0)),
            pl.BlockSpec((8 * _D, _D), lambda i: (0, 0)),
            pl.BlockSpec((8 * _D, 2 * _D), lambda i: (0, 0)),
            pl.BlockSpec((_FEW, _D), lambda i: (0, 0)),
            pl.BlockSpec((1, _D), lambda i: (0, 0)),
            pl.BlockSpec((1, 8 * _D), lambda i: (0, 0)),
        ],
        out_specs=pl.BlockSpec((_BLK, 1), lambda i: (i, 0)),
        out_shape=jax.ShapeDtypeStruct((_QROWS, 1), jnp.float32),
    )(qrows, W_ih, W_hh, sg, msg, bg2)


def kernel(query_pairs, support_pairs, symbol_emb, gcn_w_w, gcn_w_b,
           proj1_w, proj1_b, proj2_w, proj2_b, ln_a, ln_b,
           W_ih, W_hh, b_ih, b_hh):
    qidx = query_pairs.reshape(-1).astype(jnp.int32)
    rel = support_pairs[:, :, 0]
    ent = support_pairs[:, :, 1]
    sidx = jnp.concatenate([rel, ent], axis=0).astype(jnp.int32)
    sidx = jnp.pad(sidx, ((0, 0), (0, _NPAD - _NEIGH)),
                   constant_values=_NSYM)

    qrows, ssum = _sc_gather(qidx, sidx, symbol_emb)

    sg, msg = _tc_support(
        ssum, gcn_w_w, proj1_w, proj2_w,
        gcn_w_b.reshape(1, _D), proj1_b.reshape(1, 2 * _D),
        proj2_b.reshape(1, _D), ln_a.reshape(1, _D), ln_b.reshape(1, _D))

    bg2 = (b_ih + b_hh).reshape(1, 8 * _D)
    s = _tc_lstm(qrows, W_ih, W_hh, sg, msg, bg2)
    s = s.reshape(-1, 2)
    return 0.5 * (s[:, 0] + s[:, 1])
